# baseline (speedup 1.0000x reference)
import functools

import jax
import jax.numpy as jnp
import numpy as np
from jax import lax
from jax.experimental import pallas as pl
from jax.experimental.pallas import tpu as pltpu
from jax.experimental.pallas import tpu_sc as plsc

NC = 2
NS = 16
NW = NC * NS
LANES = 16
CHUNK = 128

_SC_PARAMS = pltpu.CompilerParams(
    needs_layout_passes=False, use_tc_tiling_on_sc=False)


def _sigmoid(x):
    return 1.0 / (1.0 + jnp.exp(-x))


def _sc_edge_prep(N, NP, EPT):
    mesh = plsc.VectorSubcoreMesh(core_axis_name="c", subcore_axis_name="s")

    @functools.partial(
        pl.kernel,
        out_type=(
            jax.ShapeDtypeStruct((8, NW * EPT // 8), jnp.float32),
            jax.ShapeDtypeStruct((NW, N), jnp.float32),
        ),
        mesh=mesh,
        scratch_types=[
            pltpu.VMEM((EPT,), jnp.int32),
            pltpu.VMEM((EPT,), jnp.int32),
            pltpu.VMEM((NP * 4,), jnp.float32),
            pltpu.VMEM((NP,), jnp.float32),
            pltpu.VMEM((EPT,), jnp.float32),
        ],
        compiler_params=_SC_PARAMS,
    )
    def k(row_hbm, col_hbm, coords_hbm, d2_hbm, cntp_hbm,
          row_v, col_v, xyz_v, cnt_v, d2_v):
        cid = lax.axis_index("c")
        sid = lax.axis_index("s")
        wid = sid * NC + cid
        base = wid * EPT
        pltpu.sync_copy(row_hbm.at[pl.ds(base, EPT)], row_v)
        pltpu.sync_copy(col_hbm.at[pl.ds(base, EPT)], col_v)
        pltpu.sync_copy(coords_hbm, xyz_v)
        zero16 = jnp.zeros((LANES,), jnp.float32)

        def zbody(i, carry):
            cnt_v[pl.ds(i * LANES, LANES)] = zero16
            return carry

        lax.fori_loop(0, NP // LANES, zbody, 0)
        ones = jnp.full((LANES,), 1.0, jnp.float32)
        EPTB = EPT // 8
        io = lax.iota(jnp.int32, LANES)
        cvec = (io % 8) * EPTB + io // 8

        def body(i, carry):
            for u in range(4):
                off = i * (4 * LANES) + u * LANES
                r4 = row_v[pl.ds(off, LANES)] * 4
                c4 = col_v[pl.ds(off, LANES)] * 4
                dx = plsc.load_gather(xyz_v, [r4]) - plsc.load_gather(xyz_v, [c4])
                dy = plsc.load_gather(xyz_v, [r4 + 1]) - plsc.load_gather(xyz_v, [c4 + 1])
                dz = plsc.load_gather(xyz_v, [r4 + 2]) - plsc.load_gather(xyz_v, [c4 + 2])
                plsc.store_scatter(d2_v, [cvec + (off // 8)],
                                   dx * dx + dy * dy + dz * dz)
                plsc.addupdate_scatter(cnt_v, [col_v[pl.ds(off, LANES)]], ones)
            return carry

        lax.fori_loop(0, EPT // (4 * LANES), body, 0)
        for s in range(8):
            pltpu.sync_copy(d2_v.at[pl.ds(s * EPTB, EPTB)],
                            d2_hbm.at[s].at[pl.ds(wid * EPTB, EPTB)])
        pltpu.sync_copy(cnt_v.at[pl.ds(0, N)], cntp_hbm.at[wid])

    return k


def _tc_mlp(N, NB, BLKB, EPB, HID, OUT_C):
    def body(d28_ref, cntp_ref, f_ref, r_ref, qt_ref, r8a_ref, r8b_ref,
             mblk_ref, r8at_ref, w1_ref, b1_ref, w2_ref, b2_ref, w3_ref,
             b3t_ref, msg_ref, m3b_s, c1r_s):
        @pl.when(pl.program_id(0) == 0)
        def _():
            cnt1 = jnp.sum(cntp_ref[...], axis=0, keepdims=True)
            s1 = jnp.dot(cnt1, f_ref[...],
                         preferred_element_type=jnp.float32)
            srep = jnp.dot(s1, r_ref[...],
                           preferred_element_type=jnp.float32)
            m3 = jnp.dot(qt_ref[...] * srep, w3_ref[...],
                         preferred_element_type=jnp.float32)
            m3b_s[...] = mblk_ref[...] * jnp.dot(
                jnp.dot(r8a_ref[...], m3, preferred_element_type=jnp.float32),
                r8b_ref[...], preferred_element_type=jnp.float32)
            c1 = jnp.dot(s1, b3t_ref[...],
                         preferred_element_type=jnp.float32)
            c1r_s[...] = jnp.dot(c1, r8at_ref[...],
                                 preferred_element_type=jnp.float32)

        dd8 = jnp.sqrt(d28_ref[...])
        parts = []
        for s in range(8):
            pre1 = w1_ref[...] * dd8[s:s + 1, :] + b1_ref[...]
            h = pre1 * _sigmoid(pre1)
            pre2 = jnp.dot(w2_ref[...], h,
                           preferred_element_type=jnp.float32) + b2_ref[...]
            parts.append(pre2 * _sigmoid(pre2))
        filt8 = jnp.concatenate(parts, axis=0)
        msgp = lax.dot_general(filt8, m3b_s[...], (((0,), (1,)), ((), ())),
                               preferred_element_type=jnp.float32)
        msg_ref[...] = msgp + c1r_s[...]

    IN_C = 16
    return pl.pallas_call(
        body,
        grid=(NB,),
        in_specs=[
            pl.BlockSpec((8, BLKB), lambda i: (0, i)),
            pl.BlockSpec((NW, N), lambda i: (0, 0)),
            pl.BlockSpec((N, IN_C), lambda i: (0, 0)),
            pl.BlockSpec((IN_C, OUT_C * IN_C), lambda i: (0, 0)),
            pl.BlockSpec((OUT_C, OUT_C * IN_C), lambda i: (0, 0)),
            pl.BlockSpec((8 * OUT_C, OUT_C), lambda i: (0, 0)),
            pl.BlockSpec((HID, 8 * HID), lambda i: (0, 0)),
            pl.BlockSpec((8 * OUT_C, 8 * HID), lambda i: (0, 0)),
            pl.BlockSpec((OUT_C, 8 * OUT_C), lambda i: (0, 0)),
            pl.BlockSpec((HID, 1), lambda i: (0, 0)),
            pl.BlockSpec((HID, 1), lambda i: (0, 0)),
            pl.BlockSpec((HID, HID), lambda i: (0, 0)),
            pl.BlockSpec((HID, 1), lambda i: (0, 0)),
            pl.BlockSpec((OUT_C * IN_C, HID), lambda i: (0, 0)),
            pl.BlockSpec((IN_C, OUT_C), lambda i: (0, 0)),
        ],
        out_specs=pl.BlockSpec((BLKB, 8 * OUT_C), lambda i: (i, 0)),
        out_shape=jax.ShapeDtypeStruct((EPB, 8 * OUT_C), jnp.float32),
        scratch_shapes=[
            pltpu.VMEM((8 * OUT_C, 8 * HID), jnp.float32),
            pltpu.VMEM((1, 8 * OUT_C), jnp.float32),
        ],
        compiler_params=pltpu.CompilerParams(
            dimension_semantics=("arbitrary",),
            fuse_transposed_lhs_in_matmul=True),
    )


def _sc_scatter(NP, EPT, KC, OUT_C):
    mesh = plsc.VectorSubcoreMesh(core_axis_name="c", subcore_axis_name="s")
    NPS = NP // NS

    @functools.partial(
        pl.kernel,
        out_type=jax.ShapeDtypeStruct((NC, NP, OUT_C), jnp.float32),
        mesh=mesh,
        scratch_types=[
            pltpu.VMEM((EPT, OUT_C), jnp.float32),
            pltpu.VMEM((KC, CHUNK), jnp.int32),
            pltpu.VMEM_SHARED((NP, OUT_C), jnp.float32),
            pltpu.SemaphoreType.DMA,
        ],
        compiler_params=_SC_PARAMS,
    )
    def k(msg_hbm, idx_hbm, zeros_hbm, outp_hbm, msg_v, idx_v, acc_s, sem):
        cid = lax.axis_index("c")
        sid = lax.axis_index("s")
        wid = sid * NC + cid
        pltpu.sync_copy(zeros_hbm.at[pl.ds(sid * NPS, NPS)],
                        acc_s.at[pl.ds(sid * NPS, NPS)])
        pltpu.sync_copy(msg_hbm.at[pl.ds(wid * EPT, EPT)], msg_v)
        pltpu.sync_copy(idx_hbm.at[wid], idx_v)
        plsc.subcore_barrier()

        def fire(j, carry):
            pltpu.async_copy(msg_v.at[pl.ds(j * CHUNK, CHUNK)],
                             acc_s.at[idx_v.at[j]], sem, add=True)
            return carry

        lax.fori_loop(0, KC, fire, 0)

        def drain(j, carry):
            pltpu.make_async_copy(msg_v.at[pl.ds(0, CHUNK)],
                                  acc_s.at[idx_v.at[0]], sem).wait()
            return carry

        lax.fori_loop(0, KC, drain, 0)
        plsc.subcore_barrier()
        pltpu.sync_copy(acc_s.at[pl.ds(sid * NPS, NPS)],
                        outp_hbm.at[cid].at[pl.ds(sid * NPS, NPS)])

    return k


def _tc_combine(NPR):
    def body(p_ref, o_ref):
        o_ref[...] = p_ref[0] + p_ref[1]

    return pl.pallas_call(
        body,
        in_specs=[pl.BlockSpec((NC, NPR, 128), lambda: (0, 0, 0))],
        out_specs=pl.BlockSpec((NPR, 128), lambda: (0, 0)),
        out_shape=jax.ShapeDtypeStruct((NPR, 128), jnp.float32),
    )


def kernel(features, coords, edge_index, W1, b1, W2, b2, W3, b3):
    N, IN_C = features.shape
    E = edge_index.shape[1]
    HID = W2.shape[0]
    OUT_C = W3.shape[0] // IN_C

    NP = ((N + 1 + 127) // 128) * 128
    EPT = -(-E // NW)
    EPT = ((EPT + 255) // 256) * 256
    EP = EPT * NW
    KC = EPT // CHUNK
    BLK = 32768
    NB = EP // BLK

    row_p = jnp.pad(edge_index[0], (0, EP - E), constant_values=N)
    col_p = jnp.pad(edge_index[1], (0, EP - E), constant_values=N)
    coords_f = jnp.pad(coords, ((0, NP - N), (0, 1))).reshape(NP * 4)
    R = jnp.asarray(np.tile(np.eye(IN_C, dtype=np.float32), (1, OUT_C)))
    QT = jnp.asarray(np.repeat(np.eye(OUT_C, dtype=np.float32), IN_C, axis=1))
    R8a = jnp.asarray(np.tile(np.eye(OUT_C, dtype=np.float32), (8, 1)))
    R8b = jnp.asarray(np.tile(np.eye(HID, dtype=np.float32), (1, 8)))
    MASKblk = jnp.asarray(np.kron(np.eye(8, dtype=np.float32),
                                  np.ones((OUT_C, HID), dtype=np.float32)))
    R8aT = jnp.asarray(np.tile(np.eye(OUT_C, dtype=np.float32), (1, 8)))
    b3rT = b3.reshape(OUT_C, IN_C).T
    b1c = b1.reshape(HID, 1)
    b2c = b2.reshape(HID, 1)

    d28, cntp = _sc_edge_prep(N, NP, EPT)(row_p, col_p, coords_f)
    EPB = EP // 8
    BLKB = BLK // 8
    NB = EPB // BLKB
    msgp = _tc_mlp(N, NB, BLKB, EPB, HID, OUT_C)(
        d28, cntp, features, R, QT, R8a, R8b, MASKblk, R8aT,
        W1, b1c, W2, b2c, W3, b3rT)
    msg = msgp.reshape(EP, OUT_C)
    idx3d = row_p.reshape(NW, KC, CHUNK)
    zeros_out = jnp.zeros((NP, OUT_C), jnp.float32)
    outp = _sc_scatter(NP, EPT, KC, OUT_C)(msg, idx3d, zeros_out)
    NPR = NP * OUT_C // 128
    out2 = _tc_combine(NPR)(outp.reshape(NC, NPR, 128))
    return out2.reshape(NP, OUT_C)[:N]


# speedup vs baseline: 1.0336x; 1.0336x over previous
import functools

import jax
import jax.numpy as jnp
import numpy as np
from jax import lax
from jax.experimental import pallas as pl
from jax.experimental.pallas import tpu as pltpu
from jax.experimental.pallas import tpu_sc as plsc

NC = 2
NS = 16
NW = NC * NS
LANES = 16
CHUNK = 128

_SC_PARAMS = pltpu.CompilerParams(
    needs_layout_passes=False, use_tc_tiling_on_sc=False)


def _sigmoid(x):
    return 1.0 / (1.0 + jnp.exp(-x))


def _sc_edge_prep(N, NP, EPT):
    mesh = plsc.VectorSubcoreMesh(core_axis_name="c", subcore_axis_name="s")

    @functools.partial(
        pl.kernel,
        out_type=(
            jax.ShapeDtypeStruct((8, NW * EPT // 8), jnp.float32),
            jax.ShapeDtypeStruct((NW, N), jnp.float32),
            jax.ShapeDtypeStruct((NW * EPT,), jnp.int32),
        ),
        mesh=mesh,
        scratch_types=[
            pltpu.VMEM((EPT,), jnp.int32),
            pltpu.VMEM((EPT,), jnp.int32),
            pltpu.VMEM((NP * 4,), jnp.float32),
            pltpu.VMEM((NP,), jnp.float32),
            pltpu.VMEM((EPT,), jnp.float32),
        ],
        compiler_params=_SC_PARAMS,
    )
    def k(ei_hbm, coords_hbm, d2_hbm, cntp_hbm, rowp_hbm,
          row_v, col_v, xyz_v, cnt_v, d2_v):
        E = ei_hbm.shape[1]
        cid = lax.axis_index("c")
        sid = lax.axis_index("s")
        wid = sid * NC + cid
        base = wid * EPT
        base_sl = jnp.minimum(base, E - EPT)
        pltpu.sync_copy(ei_hbm.at[0].at[pl.ds(base_sl, EPT)], row_v)
        pltpu.sync_copy(ei_hbm.at[1].at[pl.ds(base_sl, EPT)], col_v)
        pltpu.sync_copy(coords_hbm, xyz_v)
        OV = NW * EPT - E
        sent = jnp.full((LANES,), N, jnp.int32)

        @pl.when(wid == NW - 1)
        def _():
            def sbody(i, carry):
                row_v[pl.ds(i * LANES, LANES)] = sent
                col_v[pl.ds(i * LANES, LANES)] = sent
                return carry

            lax.fori_loop(0, OV // LANES, sbody, 0)
        zero16 = jnp.zeros((LANES,), jnp.float32)

        def zbody(i, carry):
            cnt_v[pl.ds(i * LANES, LANES)] = zero16
            return carry

        lax.fori_loop(0, NP // LANES, zbody, 0)
        ones = jnp.full((LANES,), 1.0, jnp.float32)
        EPTB = EPT // 8
        io = lax.iota(jnp.int32, LANES)
        cvec = (io % 8) * EPTB + io // 8

        def body(i, carry):
            for u in range(4):
                off = i * (4 * LANES) + u * LANES
                r4 = row_v[pl.ds(off, LANES)] * 4
                c4 = col_v[pl.ds(off, LANES)] * 4
                dx = plsc.load_gather(xyz_v, [r4]) - plsc.load_gather(xyz_v, [c4])
                dy = plsc.load_gather(xyz_v, [r4 + 1]) - plsc.load_gather(xyz_v, [c4 + 1])
                dz = plsc.load_gather(xyz_v, [r4 + 2]) - plsc.load_gather(xyz_v, [c4 + 2])
                plsc.store_scatter(d2_v, [cvec + (off // 8)],
                                   dx * dx + dy * dy + dz * dz)
                plsc.addupdate_scatter(cnt_v, [col_v[pl.ds(off, LANES)]], ones)
            return carry

        lax.fori_loop(0, EPT // (4 * LANES), body, 0)
        for s in range(8):
            pltpu.sync_copy(d2_v.at[pl.ds(s * EPTB, EPTB)],
                            d2_hbm.at[s].at[pl.ds(wid * EPTB, EPTB)])
        pltpu.sync_copy(cnt_v.at[pl.ds(0, N)], cntp_hbm.at[wid])
        pltpu.sync_copy(row_v, rowp_hbm.at[pl.ds(base, EPT)])

    return k


def _tc_mlp(N, NB, BLKB, EPB, HID, OUT_C):
    def body(d28_ref, cntp_ref, f_ref, r_ref, qt_ref, r8a_ref, r8b_ref,
             mblk_ref, r8at_ref, w1_ref, b1_ref, w2_ref, b2_ref, w3_ref,
             b3t_ref, msg_ref, m3b_s, c1r_s):
        @pl.when(pl.program_id(0) == 0)
        def _():
            cnt1 = jnp.sum(cntp_ref[...], axis=0, keepdims=True)
            s1 = jnp.dot(cnt1, f_ref[...],
                         preferred_element_type=jnp.float32)
            srep = jnp.dot(s1, r_ref[...],
                           preferred_element_type=jnp.float32)
            m3 = jnp.dot(qt_ref[...] * srep, w3_ref[...],
                         preferred_element_type=jnp.float32)
            m3b_s[...] = mblk_ref[...] * jnp.dot(
                jnp.dot(r8a_ref[...], m3, preferred_element_type=jnp.float32),
                r8b_ref[...], preferred_element_type=jnp.float32)
            c1 = jnp.dot(s1, b3t_ref[...],
                         preferred_element_type=jnp.float32)
            c1r_s[...] = jnp.dot(c1, r8at_ref[...],
                                 preferred_element_type=jnp.float32)

        dd8 = jnp.sqrt(d28_ref[...])
        parts = []
        for s in range(8):
            pre1 = w1_ref[...] * dd8[s:s + 1, :] + b1_ref[...]
            h = pre1 * _sigmoid(pre1)
            pre2 = jnp.dot(w2_ref[...], h,
                           preferred_element_type=jnp.float32) + b2_ref[...]
            parts.append(pre2 * _sigmoid(pre2))
        filt8 = jnp.concatenate(parts, axis=0)
        msgp = lax.dot_general(filt8, m3b_s[...], (((0,), (1,)), ((), ())),
                               preferred_element_type=jnp.float32)
        msg_ref[...] = msgp + c1r_s[...]

    IN_C = 16
    return pl.pallas_call(
        body,
        grid=(NB,),
        in_specs=[
            pl.BlockSpec((8, BLKB), lambda i: (0, i)),
            pl.BlockSpec((NW, N), lambda i: (0, 0)),
            pl.BlockSpec((N, IN_C), lambda i: (0, 0)),
            pl.BlockSpec((IN_C, OUT_C * IN_C), lambda i: (0, 0)),
            pl.BlockSpec((OUT_C, OUT_C * IN_C), lambda i: (0, 0)),
            pl.BlockSpec((8 * OUT_C, OUT_C), lambda i: (0, 0)),
            pl.BlockSpec((HID, 8 * HID), lambda i: (0, 0)),
            pl.BlockSpec((8 * OUT_C, 8 * HID), lambda i: (0, 0)),
            pl.BlockSpec((OUT_C, 8 * OUT_C), lambda i: (0, 0)),
            pl.BlockSpec((HID, 1), lambda i: (0, 0)),
            pl.BlockSpec((HID, 1), lambda i: (0, 0)),
            pl.BlockSpec((HID, HID), lambda i: (0, 0)),
            pl.BlockSpec((HID, 1), lambda i: (0, 0)),
            pl.BlockSpec((OUT_C * IN_C, HID), lambda i: (0, 0)),
            pl.BlockSpec((IN_C, OUT_C), lambda i: (0, 0)),
        ],
        out_specs=pl.BlockSpec((BLKB, 8 * OUT_C), lambda i: (i, 0)),
        out_shape=jax.ShapeDtypeStruct((EPB, 8 * OUT_C), jnp.float32),
        scratch_shapes=[
            pltpu.VMEM((8 * OUT_C, 8 * HID), jnp.float32),
            pltpu.VMEM((1, 8 * OUT_C), jnp.float32),
        ],
        compiler_params=pltpu.CompilerParams(
            dimension_semantics=("arbitrary",),
            fuse_transposed_lhs_in_matmul=True),
    )


def _sc_scatter(NP, EPT, KC, OUT_C):
    mesh = plsc.VectorSubcoreMesh(core_axis_name="c", subcore_axis_name="s")
    NPS = NP // NS

    @functools.partial(
        pl.kernel,
        out_type=jax.ShapeDtypeStruct((NC, NP, OUT_C), jnp.float32),
        mesh=mesh,
        scratch_types=[
            pltpu.VMEM((EPT, OUT_C), jnp.float32),
            pltpu.VMEM((KC, CHUNK), jnp.int32),
            pltpu.VMEM_SHARED((NP, OUT_C), jnp.float32),
            pltpu.SemaphoreType.DMA,
        ],
        compiler_params=_SC_PARAMS,
    )
    def k(msg_hbm, idx_hbm, zeros_hbm, outp_hbm, msg_v, idx_v, acc_s, sem):
        cid = lax.axis_index("c")
        sid = lax.axis_index("s")
        wid = sid * NC + cid
        pltpu.sync_copy(zeros_hbm.at[pl.ds(sid * NPS, NPS)],
                        acc_s.at[pl.ds(sid * NPS, NPS)])
        pltpu.sync_copy(msg_hbm.at[pl.ds(wid * EPT, EPT)], msg_v)
        pltpu.sync_copy(idx_hbm.at[wid], idx_v)
        plsc.subcore_barrier()

        def fire(j, carry):
            pltpu.async_copy(msg_v.at[pl.ds(j * CHUNK, CHUNK)],
                             acc_s.at[idx_v.at[j]], sem, add=True)
            return carry

        lax.fori_loop(0, KC, fire, 0)

        def drain(j, carry):
            pltpu.make_async_copy(msg_v.at[pl.ds(0, CHUNK)],
                                  acc_s.at[idx_v.at[0]], sem).wait()
            return carry

        lax.fori_loop(0, KC, drain, 0)
        plsc.subcore_barrier()
        pltpu.sync_copy(acc_s.at[pl.ds(sid * NPS, NPS)],
                        outp_hbm.at[cid].at[pl.ds(sid * NPS, NPS)])

    return k


def _tc_combine(NPR):
    def body(p_ref, o_ref):
        o_ref[...] = p_ref[0] + p_ref[1]

    return pl.pallas_call(
        body,
        in_specs=[pl.BlockSpec((NC, NPR, 128), lambda: (0, 0, 0))],
        out_specs=pl.BlockSpec((NPR, 128), lambda: (0, 0)),
        out_shape=jax.ShapeDtypeStruct((NPR, 128), jnp.float32),
    )


def kernel(features, coords, edge_index, W1, b1, W2, b2, W3, b3):
    N, IN_C = features.shape
    E = edge_index.shape[1]
    HID = W2.shape[0]
    OUT_C = W3.shape[0] // IN_C

    NP = ((N + 1 + 127) // 128) * 128
    EPT = -(-E // NW)
    EPT = ((EPT + 255) // 256) * 256
    EP = EPT * NW
    KC = EPT // CHUNK
    BLK = 32768
    NB = EP // BLK

    coords_f = jnp.pad(coords, ((0, NP - N), (0, 1))).reshape(NP * 4)
    R = jnp.asarray(np.tile(np.eye(IN_C, dtype=np.float32), (1, OUT_C)))
    QT = jnp.asarray(np.repeat(np.eye(OUT_C, dtype=np.float32), IN_C, axis=1))
    R8a = jnp.asarray(np.tile(np.eye(OUT_C, dtype=np.float32), (8, 1)))
    R8b = jnp.asarray(np.tile(np.eye(HID, dtype=np.float32), (1, 8)))
    MASKblk = jnp.asarray(np.kron(np.eye(8, dtype=np.float32),
                                  np.ones((OUT_C, HID), dtype=np.float32)))
    R8aT = jnp.asarray(np.tile(np.eye(OUT_C, dtype=np.float32), (1, 8)))
    b3rT = b3.reshape(OUT_C, IN_C).T
    b1c = b1.reshape(HID, 1)
    b2c = b2.reshape(HID, 1)

    d28, cntp, rowp = _sc_edge_prep(N, NP, EPT)(edge_index, coords_f)
    EPB = EP // 8
    BLKB = BLK // 8
    NB = EPB // BLKB
    msgp = _tc_mlp(N, NB, BLKB, EPB, HID, OUT_C)(
        d28, cntp, features, R, QT, R8a, R8b, MASKblk, R8aT,
        W1, b1c, W2, b2c, W3, b3rT)
    msg = msgp.reshape(EP, OUT_C)
    idx3d = rowp.reshape(NW, KC, CHUNK)
    zeros_out = jnp.zeros((NP, OUT_C), jnp.float32)
    outp = _sc_scatter(NP, EPT, KC, OUT_C)(msg, idx3d, zeros_out)
    NPR = NP * OUT_C // 128
    out2 = _tc_combine(NPR)(outp.reshape(NC, NPR, 128))
    return out2.reshape(NP, OUT_C)[:N]
